# baseline (speedup 1.0000x reference)
import functools

import jax
import jax.numpy as jnp
from jax import lax
from jax.experimental import pallas as pl
from jax.experimental.pallas import tpu as pltpu
from jax.experimental.pallas import tpu_sc as plsc

_NC, _NS, _VEC = 2, 16, 16
_NW = _NC * _NS
_CH = 128


def _comb2_body(pos2_ref, tab_ref, out_ref):
    c = tab_ref.shape[1]
    pe = pos2_ref[:, :c]
    po = pos2_ref[:, c:]
    t0 = tab_ref[0, :][None, :]
    t1 = tab_ref[1, :][None, :]
    out_ref[0, :, :c] = pe + t0
    out_ref[0, :, c:] = po + t0
    out_ref[1, :, :c] = pe + t0
    out_ref[1, :, c:] = po + t1
    out_ref[2, :, :c] = pe + t1
    out_ref[2, :, c:] = po + t0
    out_ref[3, :, :c] = pe + t1
    out_ref[3, :, c:] = po + t1


def _build_comb2(pos2, mask_table, th, c):
    return pl.pallas_call(
        _comb2_body,
        in_specs=[
            pl.BlockSpec((th, 2 * c), lambda: (0, 0)),
            pl.BlockSpec((2, c), lambda: (0, 0)),
        ],
        out_specs=pl.BlockSpec((4, th, 2 * c), lambda: (0, 0, 0)),
        out_shape=jax.ShapeDtypeStruct((4, th, 2 * c), jnp.float32),
    )(pos2, mask_table)


def _make_sc_gather(tok, t, c):
    th = t // 2
    pairs = tok // 2
    per_w = tok // _NW
    per_wp = per_w // 2
    nsc = per_wp // _CH
    mesh = plsc.VectorSubcoreMesh(
        core_axis_name="c", subcore_axis_name="s",
        num_cores=_NC, num_subcores=_NS,
    )

    @functools.partial(
        pl.kernel,
        out_type=jax.ShapeDtypeStruct((pairs, 2, c), jnp.float32),
        mesh=mesh,
        scratch_types=[
            pltpu.VMEM_SHARED((4 * th, 2, c), jnp.float32),
            pltpu.VMEM((per_w + _VEC,), jnp.int32),
            pltpu.VMEM((nsc, _CH), jnp.int32),
            pltpu.VMEM((2, _CH, 2, c), jnp.float32),
            pltpu.SemaphoreType.DMA,
            pltpu.SemaphoreType.DMA,
        ],
    )
    def sc_gather(comb2_hbm, mask_hbm, out_hbm,
                  comb2_sh, mask_v, idx_v, bufs, sem0, sem1):
        sid = lax.axis_index("s")
        wid = sid * _NC + lax.axis_index("c")
        base = wid * per_w
        pbase = wid * per_wp

        @pl.when(sid == 0)
        def _():
            pltpu.sync_copy(comb2_hbm, comb2_sh)

        pltpu.sync_copy(mask_hbm.at[pl.ds(base, per_w)],
                        mask_v.at[pl.ds(0, per_w)])

        lanes = lax.iota(jnp.int32, _VEC)
        even_perm = lax.rem(lanes * 2, _VEC)
        dnums = lax.GatherDimensionNumbers(
            offset_dims=(), collapsed_slice_dims=(0,), start_index_map=(0,))

        def permute(x):
            return lax.gather(
                x, even_perm[:, None], dnums, slice_sizes=(1,),
                mode=lax.GatherScatterMode.PROMISE_IN_BOUNDS)

        def idx_row(j, _):
            def idx_vec(v, _):
                o = j * 2 * _CH + v * 2 * _VEC
                a = mask_v[pl.ds(o, _VEC)]
                ash = mask_v[pl.ds(o + 1, _VEC)]
                b2 = mask_v[pl.ds(o + _VEC, _VEC)]
                bsh = mask_v[pl.ds(o + _VEC + 1, _VEC)]
                ca = 2 * a + ash
                cb = 2 * b2 + bsh
                ga = permute(ca)
                gb = permute(cb)
                code = jnp.where(lanes < 8, ga, gb)
                pp = pbase + j * _CH + v * _VEC + lanes
                tt = lax.rem(pp, th)
                idx_v[j, pl.ds(v * _VEC, _VEC)] = code * th + tt
                return 0
            return lax.fori_loop(0, _CH // _VEC, idx_vec, 0)

        lax.fori_loop(0, 2, idx_row, 0)
        plsc.subcore_barrier()

        b0 = bufs.at[0]
        b1 = bufs.at[1]

        def fire(s, buf, sem):
            pltpu.async_copy(comb2_sh.at[idx_v.at[s]], buf, sem)

        def drain(buf, sem):
            pltpu.make_async_copy(out_hbm.at[pl.ds(0, _CH)], buf, sem).wait()

        def scatter(s, buf):
            pltpu.sync_copy(buf, out_hbm.at[pl.ds(pbase + s * _CH, _CH)])

        fire(0, b0, sem0)
        fire(1, b1, sem1)

        lax.fori_loop(2, nsc, idx_row, 0)

        def pair(g, _):
            s0 = 2 * g
            s1 = s0 + 1
            drain(b0, sem0)
            scatter(s0, b0)

            @pl.when(s0 + 2 < nsc)
            def _():
                fire(s0 + 2, b0, sem0)

            drain(b1, sem1)
            scatter(s1, b1)

            @pl.when(s1 + 2 < nsc)
            def _():
                fire(s1 + 2, b1, sem1)
            return 0

        lax.fori_loop(0, nsc // 2, pair, 0)

        drain(b0, sem0)
        scatter(nsc - 1, b0)

    return sc_gather


def kernel(x, mask, pos_emb, mask_table):
    b, t, c = x.shape
    tok = b * t
    pos2 = pos_emb[0, :t, :].reshape(t // 2, 2 * c)
    m_flat = mask.astype(jnp.int32).reshape(tok)
    comb2 = _build_comb2(pos2, mask_table, t // 2, c).reshape(2 * t, 2, c)
    out = _make_sc_gather(tok, t, c)(comb2, m_flat)
    return out.reshape(b, t, c)


# speedup vs baseline: 1.0553x; 1.0553x over previous
import functools

import jax
import jax.numpy as jnp
from jax import lax
from jax.experimental import pallas as pl
from jax.experimental.pallas import tpu as pltpu
from jax.experimental.pallas import tpu_sc as plsc

_NC, _NS, _VEC = 2, 16, 16
_NW = _NC * _NS
_CH = 128
_SCH = 2 * _CH


def _comb_body(pos_ref, tab_ref, out_ref):
    out_ref[0] = pos_ref[...] + tab_ref[0, :][None, :]
    out_ref[1] = pos_ref[...] + tab_ref[1, :][None, :]


def _build_comb(pos, mask_table, t, c):
    return pl.pallas_call(
        _comb_body,
        in_specs=[
            pl.BlockSpec((t, c), lambda: (0, 0)),
            pl.BlockSpec((2, c), lambda: (0, 0)),
        ],
        out_specs=pl.BlockSpec((2, t, c), lambda: (0, 0, 0)),
        out_shape=jax.ShapeDtypeStruct((2, t, c), jnp.float32),
    )(pos, mask_table)


def _make_sc_gather(tok, t, c):
    per_w = tok // _NW
    nch = per_w // _CH
    nsc = per_w // _SCH
    mesh = plsc.VectorSubcoreMesh(
        core_axis_name="c", subcore_axis_name="s",
        num_cores=_NC, num_subcores=_NS,
    )

    @functools.partial(
        pl.kernel,
        out_type=jax.ShapeDtypeStruct((tok, c), jnp.float32),
        mesh=mesh,
        scratch_types=[
            pltpu.VMEM_SHARED((2 * t, c), jnp.float32),
            pltpu.VMEM((per_w,), jnp.int32),
            pltpu.VMEM((nch, _CH), jnp.int32),
            pltpu.VMEM((2, _SCH, c), jnp.float32),
            pltpu.SemaphoreType.DMA,
            pltpu.SemaphoreType.DMA,
        ],
    )
    def sc_gather(comb_hbm, mask_hbm, out_hbm,
                  comb_sh, mask_v, idx_v, bufs, sem0, sem1):
        sid = lax.axis_index("s")
        wid = sid * _NC + lax.axis_index("c")
        base = wid * per_w

        @pl.when(sid == 0)
        def _():
            pltpu.sync_copy(comb_hbm, comb_sh)

        pltpu.sync_copy(mask_hbm.at[pl.ds(base, per_w)], mask_v)

        lanes = lax.iota(jnp.int32, _VEC)

        def idx_row(j, _):
            def idx_vec(v, _):
                p = j * _CH + v * _VEC
                m = mask_v[pl.ds(p, _VEC)]
                tpos = lax.rem(base + p + lanes, t)
                idx_v[j, pl.ds(v * _VEC, _VEC)] = m * t + tpos
                return 0
            return lax.fori_loop(0, _CH // _VEC, idx_vec, 0)

        lax.fori_loop(0, 4, idx_row, 0)
        plsc.subcore_barrier()

        b0 = bufs.at[0]
        b1 = bufs.at[1]

        def fire(s, buf, sem):
            pltpu.async_copy(comb_sh.at[idx_v.at[2 * s]],
                             buf.at[pl.ds(0, _CH)], sem)
            pltpu.async_copy(comb_sh.at[idx_v.at[2 * s + 1]],
                             buf.at[pl.ds(_CH, _CH)], sem)

        def drain(buf, sem):
            pltpu.make_async_copy(out_hbm.at[pl.ds(0, _SCH)], buf, sem).wait()

        def scatter(s, buf):
            pltpu.sync_copy(buf, out_hbm.at[pl.ds(base + s * _SCH, _SCH)])

        fire(0, b0, sem0)
        fire(1, b1, sem1)

        lax.fori_loop(4, nch, idx_row, 0)

        def pair(g, _):
            s0 = 2 * g
            s1 = s0 + 1
            drain(b0, sem0)
            scatter(s0, b0)

            @pl.when(s0 + 2 < nsc)
            def _():
                fire(s0 + 2, b0, sem0)

            drain(b1, sem1)
            scatter(s1, b1)

            @pl.when(s1 + 2 < nsc)
            def _():
                fire(s1 + 2, b1, sem1)
            return 0

        lax.fori_loop(0, nsc // 2, pair, 0)

        drain(b0, sem0)
        scatter(nsc - 1, b0)

    return sc_gather


def kernel(x, mask, pos_emb, mask_table):
    b, t, c = x.shape
    tok = b * t
    pos = pos_emb[0, :t, :]
    m_flat = mask.astype(jnp.int32).reshape(tok)
    comb = _build_comb(pos, mask_table, t, c).reshape(2 * t, c)
    out = _make_sc_gather(tok, t, c)(comb, m_flat)
    return out.reshape(b, t, c)
